# baseline (speedup 1.0000x reference)
import functools

import jax
import jax.numpy as jnp
import numpy as np
from jax import lax
from jax.experimental import pallas as pl
from jax.experimental.pallas import tpu as pltpu
from jax.experimental.pallas import tpu_sc as plsc

_NUM_CORES = 2
_NUM_SUBCORES = 16
_LANES = 16
_NUM_WORKERS = _NUM_CORES * _NUM_SUBCORES

_B, _D, _G = 16384, 11, 10
_ROWS = _B // _NUM_WORKERS
_STEPS = _ROWS // _LANES

_T2_BITS = (0x3C23D70A, 0x3D23D70A, 0x3DB851EC, 0x3E23D70A, 0x3E800000,
            0x3EB851EC, 0x3EFAE146, 0x3F23D70A, 0x3F4F5C2A)
_T2 = tuple(float(np.uint32(b).view(np.float32)) for b in _T2_BITS)


def _vfrho_body(z1_hbm, z2_hbm, dg_hbm, out_hbm, z1_v, z2_v, dg_v, out_v):
    wid = lax.axis_index("s") * _NUM_CORES + lax.axis_index("c")
    base = wid * _ROWS
    pltpu.sync_copy(z1_hbm.at[pl.ds(base * _D, _ROWS * _D)], z1_v)
    pltpu.sync_copy(z2_hbm.at[pl.ds(base * _D, _ROWS * _D)], z2_v)
    pltpu.sync_copy(dg_hbm.at[pl.ds(base * _G, _ROWS * _G)], dg_v)

    lane = lax.iota(jnp.int32, _LANES)
    t2 = [jnp.full((_LANES,), v, jnp.float32) for v in _T2]

    def step(i, carry):
        rows = lane + i * _LANES
        zoff = rows * _D
        x1 = plsc.load_gather(z1_v, [zoff])
        x2 = plsc.load_gather(z2_v, [zoff])
        y1 = plsc.load_gather(z1_v, [zoff + 2])
        y2 = plsc.load_gather(z2_v, [zoff + 2])
        dx = x2 - x1
        dy = y2 - y1
        r2 = dx * dx + dy * dy
        bucket = jnp.zeros((_LANES,), jnp.int32)
        for c in t2:
            bucket = bucket + (r2 >= c).astype(jnp.int32)
        val = plsc.load_gather(dg_v, [rows * _G + bucket])
        out_v[pl.ds(i * _LANES, _LANES)] = val
        return carry

    lax.fori_loop(0, _STEPS, step, 0)
    pltpu.sync_copy(out_v, out_hbm.at[pl.ds(base, _ROWS)])


_vfrho_sc = functools.partial(
    pl.kernel,
    out_type=jax.ShapeDtypeStruct((_B,), jnp.float32),
    mesh=plsc.VectorSubcoreMesh(core_axis_name="c", subcore_axis_name="s"),
    compiler_params=pltpu.CompilerParams(needs_layout_passes=False),
    scratch_types=[
        pltpu.VMEM((_ROWS * _D,), jnp.float32),
        pltpu.VMEM((_ROWS * _D,), jnp.float32),
        pltpu.VMEM((_ROWS * _G,), jnp.float32),
        pltpu.VMEM((_ROWS,), jnp.float32),
    ],
)(_vfrho_body)


def kernel(z_1, z_2, dist_grade):
    return _vfrho_sc(z_1.reshape(-1), z_2.reshape(-1), dist_grade.reshape(-1))
